# baseline (speedup 1.0000x reference)
import functools

import jax
import jax.numpy as jnp
from jax import lax
from jax.experimental import pallas as pl
from jax.experimental.pallas import tpu as pltpu
from jax.experimental.pallas import tpu_sc as plsc

N = 10000
E = 320000
D = 128
ED = 16
L = 4
G = 64

NC = 2
NS = 16
NW = NC * NS

CH = 128
KPW = 80
EPW = KPW * CH
E_PAD = NW * EPW
N_PAD = 10240
RPT = N_PAD // NS

BN = 2000
BE = 4096
NBN = N // BN


def _gelu(h):
    return 0.5 * h * (1.0 + lax.erf(h * 0.7071067811865476))


def _ln(h, g, b, eps=1e-5):
    mu = jnp.mean(h, axis=-1, keepdims=True)
    c = h - mu
    var = jnp.mean(c * c, axis=-1, keepdims=True)
    return c * jax.lax.rsqrt(var + eps) * g + b


def _onehot_t(batch_row, nrows):
    ids = lax.broadcasted_iota(jnp.int32, (G, nrows), 0)
    return (ids == jnp.broadcast_to(batch_row[None, :], (G, nrows))).astype(jnp.float32)


def _embed_body(x_ref, w_ref, b_ref, g_ref, be_ref, batch_ref, h_ref, cnt_ref):
    i = pl.program_id(0)
    h = jnp.dot(x_ref[...], w_ref[...], preferred_element_type=jnp.float32)
    h = _gelu(_ln(h + b_ref[...], g_ref[...], be_ref[...]))
    h_ref[...] = h
    oht = _onehot_t(batch_ref[0, 0, :], BN)
    cnt = jnp.sum(oht, axis=1, keepdims=True)

    @pl.when(i == 0)
    def _():
        cnt_ref[...] = jnp.zeros_like(cnt_ref)

    cnt_ref[...] += jnp.broadcast_to(cnt, (G, D))


def _embed_call(x, w, b, g, be, batch3):
    return pl.pallas_call(
        _embed_body,
        grid=(NBN,),
        in_specs=[
            pl.BlockSpec((BN, D), lambda i: (i, 0)),
            pl.BlockSpec((D, D), lambda i: (0, 0)),
            pl.BlockSpec((1, D), lambda i: (0, 0)),
            pl.BlockSpec((1, D), lambda i: (0, 0)),
            pl.BlockSpec((1, D), lambda i: (0, 0)),
            pl.BlockSpec((1, 1, BN), lambda i: (i, 0, 0)),
        ],
        out_specs=[
            pl.BlockSpec((BN, D), lambda i: (i, 0)),
            pl.BlockSpec((G, D), lambda i: (0, 0)),
        ],
        out_shape=[
            jax.ShapeDtypeStruct((N, D), jnp.float32),
            jax.ShapeDtypeStruct((G, D), jnp.float32),
        ],
    )(x, w, b, g, be, batch3)


def _msg_body(hg_ref, ea_ref, w1h_ref, w1e_ref, b1_ref, g1_ref, be1_ref,
              w2_ref, b2_ref, m_ref):
    m1 = jnp.dot(hg_ref[...].astype(jnp.bfloat16), w1h_ref[...],
                 preferred_element_type=jnp.float32)
    m1 = m1 + jnp.dot(ea_ref[...], w1e_ref[...], preferred_element_type=jnp.float32)
    m1 = _gelu(_ln(m1 + b1_ref[...], g1_ref[...], be1_ref[...]))
    m_ref[...] = jnp.dot(m1.astype(jnp.bfloat16), w2_ref[...],
                         preferred_element_type=jnp.float32) + b2_ref[...]


def _msg_call(hg, ea, w1h, w1e, b1, g1, be1, w2, b2):
    return pl.pallas_call(
        _msg_body,
        grid=(E_PAD // BE,),
        in_specs=[
            pl.BlockSpec((BE, D), lambda i: (i, 0)),
            pl.BlockSpec((BE, ED), lambda i: (i, 0)),
            pl.BlockSpec((D, 2 * D), lambda i: (0, 0)),
            pl.BlockSpec((ED, 2 * D), lambda i: (0, 0)),
            pl.BlockSpec((1, 2 * D), lambda i: (0, 0)),
            pl.BlockSpec((1, 2 * D), lambda i: (0, 0)),
            pl.BlockSpec((1, 2 * D), lambda i: (0, 0)),
            pl.BlockSpec((2 * D, D), lambda i: (0, 0)),
            pl.BlockSpec((1, D), lambda i: (0, 0)),
        ],
        out_specs=pl.BlockSpec((BE, D), lambda i: (i, 0)),
        out_shape=jax.ShapeDtypeStruct((E_PAD, D), jnp.float32),
    )(hg, ea, w1h, w1e, b1, g1, be1, w2, b2)


def _upd_body(h_ref, p_ref, wh_ref, wa_ref, b_ref, g_ref, be_ref, batch_ref,
              hn_ref, ps_ref):
    i = pl.program_id(0)
    h = h_ref[...]
    agg = p_ref[0] + p_ref[1]
    u = jnp.dot(h, wh_ref[...], preferred_element_type=jnp.float32)
    u = u + jnp.dot(agg, wa_ref[...], preferred_element_type=jnp.float32)
    u = _gelu(_ln(u + b_ref[...], g_ref[...], be_ref[...]))
    hn = u + h
    hn_ref[...] = hn
    oht = _onehot_t(batch_ref[0, 0, :], BN)

    @pl.when(i == 0)
    def _():
        ps_ref[...] = jnp.zeros_like(ps_ref)

    ps_ref[...] += jnp.dot(oht, hn, preferred_element_type=jnp.float32)


def _upd_call(h, parts, wh, wa, b, g, be, batch3):
    return pl.pallas_call(
        _upd_body,
        grid=(NBN,),
        in_specs=[
            pl.BlockSpec((BN, D), lambda i: (i, 0)),
            pl.BlockSpec((2, BN, D), lambda i: (0, i, 0)),
            pl.BlockSpec((D, D), lambda i: (0, 0)),
            pl.BlockSpec((D, D), lambda i: (0, 0)),
            pl.BlockSpec((1, D), lambda i: (0, 0)),
            pl.BlockSpec((1, D), lambda i: (0, 0)),
            pl.BlockSpec((1, D), lambda i: (0, 0)),
            pl.BlockSpec((1, 1, BN), lambda i: (i, 0, 0)),
        ],
        out_specs=[
            pl.BlockSpec((BN, D), lambda i: (i, 0)),
            pl.BlockSpec((G, D), lambda i: (0, 0)),
        ],
        out_shape=[
            jax.ShapeDtypeStruct((N, D), jnp.float32),
            jax.ShapeDtypeStruct((G, D), jnp.float32),
        ],
    )(h, parts, wh, wa, b, g, be, batch3)


def _comb_body(reps_ref, cnt_ref, w_ref, b_ref, g_ref, be_ref, out_ref):
    cnt = jnp.maximum(cnt_ref[...], 1.0)
    acc = jnp.zeros((G, D), jnp.float32)
    for l in range(L):
        acc = acc + jnp.dot(reps_ref[l] / cnt, w_ref[l],
                            preferred_element_type=jnp.float32)
    out_ref[...] = _gelu(_ln(acc + b_ref[...], g_ref[...], be_ref[...]))


def _comb_call(reps, cnt, w, b, g, be):
    return pl.pallas_call(
        _comb_body,
        out_shape=jax.ShapeDtypeStruct((G, D), jnp.float32),
    )(reps, cnt, w, b, g, be)


@functools.cache
def _sc_mesh():
    return plsc.VectorSubcoreMesh(core_axis_name="c", subcore_axis_name="s",
                                  num_cores=NC, num_subcores=NS)


_HSTRIPE = 632


def _gather_body(h_hbm, idx_hbm, out_hbm, idx_v, b0, b1, tab_sh,
                 sg0, sg1, sw0, sw1):
    cid = lax.axis_index("c")
    sid = lax.axis_index("s")
    wid = sid * NC + cid

    @pl.when(sid < NS - 1)
    def _():
        pltpu.sync_copy(h_hbm.at[pl.ds(sid * _HSTRIPE, _HSTRIPE)],
                        tab_sh.at[pl.ds(sid * _HSTRIPE, _HSTRIPE)])

    @pl.when(sid == NS - 1)
    def _():
        pltpu.sync_copy(h_hbm.at[pl.ds((NS - 1) * _HSTRIPE, N - (NS - 1) * _HSTRIPE)],
                        tab_sh.at[pl.ds((NS - 1) * _HSTRIPE, N - (NS - 1) * _HSTRIPE)])

    pltpu.sync_copy(idx_hbm.at[pl.ds(wid * KPW, KPW)], idx_v)
    plsc.subcore_barrier()

    def fg(c, buf, sem):
        pltpu.async_copy(tab_sh.at[idx_v.at[c]], buf, sem)

    def wg(buf, sem):
        pltpu.make_async_copy(tab_sh.at[pl.ds(0, CH)], buf, sem).wait()

    def fw(c, buf, sem):
        pltpu.async_copy(buf, out_hbm.at[pl.ds((wid * KPW + c) * CH, CH)], sem)

    def ww(buf, sem):
        pltpu.make_async_copy(buf, out_hbm.at[pl.ds(0, CH)], sem).wait()

    ng = KPW // 2
    fg(0, b0, sg0)

    def body(k, _):
        c = 2 * k
        wg(b0, sg0)

        @pl.when(k > 0)
        def _():
            ww(b1, sw1)

        fg(c + 1, b1, sg1)
        fw(c, b0, sw0)
        wg(b1, sg1)
        ww(b0, sw0)

        @pl.when(k < ng - 1)
        def _():
            fg(c + 2, b0, sg0)

        fw(c + 1, b1, sw1)
        return 0

    lax.fori_loop(0, ng, body, 0)
    ww(b1, sw1)


def _sc_gather(h, idx2):
    return pl.kernel(
        _gather_body,
        out_type=jax.ShapeDtypeStruct((E_PAD, D), jnp.float32),
        mesh=_sc_mesh(),
        scratch_types=[
            pltpu.VMEM((KPW, CH), jnp.int32),
            pltpu.VMEM((CH, D), jnp.float32),
            pltpu.VMEM((CH, D), jnp.float32),
            pltpu.VMEM_SHARED((N, D), jnp.float32),
            pltpu.SemaphoreType.DMA,
            pltpu.SemaphoreType.DMA,
            pltpu.SemaphoreType.DMA,
            pltpu.SemaphoreType.DMA,
        ],
    )(h, idx2)


def _scatter_body(m_hbm, idx_hbm, out_hbm, idx_v, m_b0, m_b1,
                  z_v, acc_sh, sr0, sr1, ss0, ss1):
    cid = lax.axis_index("c")
    sid = lax.axis_index("s")
    wid = sid * NC + cid
    pltpu.sync_copy(idx_hbm.at[pl.ds(wid * KPW, KPW)], idx_v)
    zrow = jnp.zeros((16,), jnp.float32)
    for r in range(8):
        for c in range(8):
            z_v[r, pl.ds(c * 16, 16)] = zrow

    def zbody(k, _):
        pltpu.sync_copy(z_v, acc_sh.at[pl.ds(sid * RPT + k * 8, 8)])
        return 0

    lax.fori_loop(0, RPT // 8, zbody, 0)
    plsc.subcore_barrier()

    def fr(c, buf, sem):
        pltpu.async_copy(m_hbm.at[pl.ds((wid * KPW + c) * CH, CH)], buf, sem)

    def wr(buf, sem):
        pltpu.make_async_copy(m_hbm.at[pl.ds(0, CH)], buf, sem).wait()

    def fs(c, buf, sem):
        pltpu.async_copy(buf, acc_sh.at[idx_v.at[c]], sem, add=True)

    def ws(buf, sem):
        pltpu.make_async_copy(buf, acc_sh.at[pl.ds(0, CH)], sem).wait()

    ng = KPW // 2
    fr(0, m_b0, sr0)

    def sbody(k, _):
        c = 2 * k
        wr(m_b0, sr0)

        @pl.when(k > 0)
        def _():
            ws(m_b1, ss1)

        fr(c + 1, m_b1, sr1)
        fs(c, m_b0, ss0)
        wr(m_b1, sr1)
        ws(m_b0, ss0)

        @pl.when(k < ng - 1)
        def _():
            fr(c + 2, m_b0, sr0)

        fs(c + 1, m_b1, ss1)
        return 0

    lax.fori_loop(0, ng, sbody, 0)
    ws(m_b1, ss1)
    plsc.subcore_barrier()
    pltpu.sync_copy(acc_sh.at[pl.ds(sid * RPT, RPT)],
                    out_hbm.at[cid, pl.ds(sid * RPT, RPT)])


def _sc_scatter(m, idx2):
    return pl.kernel(
        _scatter_body,
        out_type=jax.ShapeDtypeStruct((NC, N_PAD, D), jnp.float32),
        mesh=_sc_mesh(),
        scratch_types=[
            pltpu.VMEM((KPW, CH), jnp.int32),
            pltpu.VMEM((CH, D), jnp.float32),
            pltpu.VMEM((CH, D), jnp.float32),
            pltpu.VMEM((8, D), jnp.float32),
            pltpu.VMEM_SHARED((N_PAD, D), jnp.float32),
            pltpu.SemaphoreType.DMA,
            pltpu.SemaphoreType.DMA,
            pltpu.SemaphoreType.DMA,
            pltpu.SemaphoreType.DMA,
        ],
    )(m, idx2)


def kernel(x, edge_index, edge_attr, batch, emb_W, emb_b, emb_g, emb_beta,
           msg_W1, msg_b1, msg_g1, msg_be1, msg_W2, msg_b2,
           upd_W, upd_b, upd_g, upd_be, comb_W, comb_b, comb_g, comb_be):
    row = edge_index[0].astype(jnp.int32)
    col = edge_index[1].astype(jnp.int32)
    npad = E_PAD - E
    row2 = jnp.concatenate([row, jnp.zeros((npad,), jnp.int32)]).reshape(NW * KPW, CH)
    dummy = N + (lax.iota(jnp.int32, npad) % (N_PAD - N))
    col2 = jnp.concatenate([col, dummy]).reshape(NW * KPW, CH)
    ea = jnp.concatenate([edge_attr, jnp.zeros((npad, ED), jnp.float32)],
                         axis=0).astype(jnp.bfloat16)
    batch3 = batch.astype(jnp.int32).reshape(NBN, 1, BN)

    h, counts = _embed_call(x, emb_W, emb_b.reshape(1, D), emb_g.reshape(1, D),
                            emb_beta.reshape(1, D), batch3)
    reps = []
    for l in range(L):
        hg = _sc_gather(h, row2)
        m = _msg_call(hg, ea, msg_W1[l, :D].astype(jnp.bfloat16),
                      msg_W1[l, D:].astype(jnp.bfloat16),
                      msg_b1[l].reshape(1, 2 * D), msg_g1[l].reshape(1, 2 * D),
                      msg_be1[l].reshape(1, 2 * D), msg_W2[l].astype(jnp.bfloat16),
                      msg_b2[l].reshape(1, D))
        parts = _sc_scatter(m, col2)
        parts = lax.slice(parts, (0, 0, 0), (NC, N, D))
        h, ps = _upd_call(h, parts, upd_W[l, :D], upd_W[l, D:],
                              upd_b[l].reshape(1, D), upd_g[l].reshape(1, D),
                              upd_be[l].reshape(1, D), batch3)
        reps.append(ps)

    g = _comb_call(jnp.stack(reps), counts, comb_W.reshape(L, D, D),
                   comb_b.reshape(1, D), comb_g.reshape(1, D),
                   comb_be.reshape(1, D))
    return (g, h)


# speedup vs baseline: 1.1097x; 1.1097x over previous
import functools

import jax
import jax.numpy as jnp
from jax import lax
from jax.experimental import pallas as pl
from jax.experimental.pallas import tpu as pltpu
from jax.experimental.pallas import tpu_sc as plsc

N = 10000
E = 320000
D = 128
ED = 16
L = 4
G = 64

NC = 2
NS = 16
NW = NC * NS

CH = 128
KPW = 80
EPW = KPW * CH
E_PAD = NW * EPW
N_PAD = 10240
RPT = N_PAD // NS

S = 2
KPW_S = KPW // S
E_S = E_PAD // S
BN = 2000
BE = 4096
NBN = N // BN


def _gelu(h):
    return 0.5 * h * (1.0 + lax.erf(h * 0.7071067811865476))


def _ln(h, g, b, eps=1e-5):
    mu = jnp.mean(h, axis=-1, keepdims=True)
    c = h - mu
    var = jnp.mean(c * c, axis=-1, keepdims=True)
    return c * jax.lax.rsqrt(var + eps) * g + b


def _onehot_t(batch_row, nrows):
    ids = lax.broadcasted_iota(jnp.int32, (G, nrows), 0)
    return (ids == jnp.broadcast_to(batch_row[None, :], (G, nrows))).astype(jnp.float32)


def _embed_body(x_ref, w_ref, b_ref, g_ref, be_ref, batch_ref, h_ref, cnt_ref):
    i = pl.program_id(0)
    h = jnp.dot(x_ref[...], w_ref[...], preferred_element_type=jnp.float32)
    h = _gelu(_ln(h + b_ref[...], g_ref[...], be_ref[...]))
    h_ref[...] = h
    oht = _onehot_t(batch_ref[0, 0, :], BN)
    cnt = jnp.sum(oht, axis=1, keepdims=True)

    @pl.when(i == 0)
    def _():
        cnt_ref[...] = jnp.zeros_like(cnt_ref)

    cnt_ref[...] += jnp.broadcast_to(cnt, (G, D))


def _embed_call(x, w, b, g, be, batch3):
    return pl.pallas_call(
        _embed_body,
        grid=(NBN,),
        in_specs=[
            pl.BlockSpec((BN, D), lambda i: (i, 0)),
            pl.BlockSpec((D, D), lambda i: (0, 0)),
            pl.BlockSpec((1, D), lambda i: (0, 0)),
            pl.BlockSpec((1, D), lambda i: (0, 0)),
            pl.BlockSpec((1, D), lambda i: (0, 0)),
            pl.BlockSpec((1, 1, BN), lambda i: (i, 0, 0)),
        ],
        out_specs=[
            pl.BlockSpec((BN, D), lambda i: (i, 0)),
            pl.BlockSpec((G, D), lambda i: (0, 0)),
        ],
        out_shape=[
            jax.ShapeDtypeStruct((N, D), jnp.float32),
            jax.ShapeDtypeStruct((G, D), jnp.float32),
        ],
    )(x, w, b, g, be, batch3)


def _msg_body(hg_ref, ea_ref, w1h_ref, w1e_ref, b1_ref, g1_ref, be1_ref,
              w2_ref, b2_ref, m_ref):
    m1 = jnp.dot(hg_ref[...].astype(jnp.bfloat16), w1h_ref[...],
                 preferred_element_type=jnp.float32)
    m1 = m1 + jnp.dot(ea_ref[...], w1e_ref[...], preferred_element_type=jnp.float32)
    m1 = _gelu(_ln(m1 + b1_ref[...], g1_ref[...], be1_ref[...]))
    m_ref[...] = jnp.dot(m1.astype(jnp.bfloat16), w2_ref[...],
                         preferred_element_type=jnp.float32) + b2_ref[...]


def _msg_call(hg, ea, w1h, w1e, b1, g1, be1, w2, b2):
    return pl.pallas_call(
        _msg_body,
        grid=(hg.shape[0] // BE,),
        in_specs=[
            pl.BlockSpec((BE, D), lambda i: (i, 0)),
            pl.BlockSpec((BE, ED), lambda i: (i, 0)),
            pl.BlockSpec((D, 2 * D), lambda i: (0, 0)),
            pl.BlockSpec((ED, 2 * D), lambda i: (0, 0)),
            pl.BlockSpec((1, 2 * D), lambda i: (0, 0)),
            pl.BlockSpec((1, 2 * D), lambda i: (0, 0)),
            pl.BlockSpec((1, 2 * D), lambda i: (0, 0)),
            pl.BlockSpec((2 * D, D), lambda i: (0, 0)),
            pl.BlockSpec((1, D), lambda i: (0, 0)),
        ],
        out_specs=pl.BlockSpec((BE, D), lambda i: (i, 0)),
        out_shape=jax.ShapeDtypeStruct((hg.shape[0], D), jnp.float32),
    )(hg, ea, w1h, w1e, b1, g1, be1, w2, b2)


def _upd_body(h_ref, p_ref, wh_ref, wa_ref, b_ref, g_ref, be_ref, batch_ref,
              hn_ref, ps_ref):
    i = pl.program_id(0)
    h = h_ref[...]
    agg = p_ref[0] + p_ref[1] + p_ref[2] + p_ref[3]
    u = jnp.dot(h, wh_ref[...], preferred_element_type=jnp.float32)
    u = u + jnp.dot(agg, wa_ref[...], preferred_element_type=jnp.float32)
    u = _gelu(_ln(u + b_ref[...], g_ref[...], be_ref[...]))
    hn = u + h
    hn_ref[...] = hn
    oht = _onehot_t(batch_ref[0, 0, :], BN)

    @pl.when(i == 0)
    def _():
        ps_ref[...] = jnp.zeros_like(ps_ref)

    ps_ref[...] += jnp.dot(oht, hn, preferred_element_type=jnp.float32)


def _upd_call(h, parts, wh, wa, b, g, be, batch3):
    return pl.pallas_call(
        _upd_body,
        grid=(NBN,),
        in_specs=[
            pl.BlockSpec((BN, D), lambda i: (i, 0)),
            pl.BlockSpec((2 * S, BN, D), lambda i: (0, i, 0)),
            pl.BlockSpec((D, D), lambda i: (0, 0)),
            pl.BlockSpec((D, D), lambda i: (0, 0)),
            pl.BlockSpec((1, D), lambda i: (0, 0)),
            pl.BlockSpec((1, D), lambda i: (0, 0)),
            pl.BlockSpec((1, D), lambda i: (0, 0)),
            pl.BlockSpec((1, 1, BN), lambda i: (i, 0, 0)),
        ],
        out_specs=[
            pl.BlockSpec((BN, D), lambda i: (i, 0)),
            pl.BlockSpec((G, D), lambda i: (0, 0)),
        ],
        out_shape=[
            jax.ShapeDtypeStruct((N, D), jnp.float32),
            jax.ShapeDtypeStruct((G, D), jnp.float32),
        ],
    )(h, parts, wh, wa, b, g, be, batch3)


def _comb_body(reps_ref, cnt_ref, w_ref, b_ref, g_ref, be_ref, out_ref):
    cnt = jnp.maximum(cnt_ref[...], 1.0)
    acc = jnp.zeros((G, D), jnp.float32)
    for l in range(L):
        acc = acc + jnp.dot(reps_ref[l] / cnt, w_ref[l],
                            preferred_element_type=jnp.float32)
    out_ref[...] = _gelu(_ln(acc + b_ref[...], g_ref[...], be_ref[...]))


def _comb_call(reps, cnt, w, b, g, be):
    return pl.pallas_call(
        _comb_body,
        out_shape=jax.ShapeDtypeStruct((G, D), jnp.float32),
    )(reps, cnt, w, b, g, be)


@functools.cache
def _sc_mesh():
    return plsc.VectorSubcoreMesh(core_axis_name="c", subcore_axis_name="s",
                                  num_cores=NC, num_subcores=NS)


_HSTRIPE = 632


def _gather_body(h_hbm, idx_hbm, out_hbm, idx_v, b0, b1, tab_sh,
                 sg0, sg1, sw0, sw1, *, kpw):
    cid = lax.axis_index("c")
    sid = lax.axis_index("s")
    wid = sid * NC + cid

    @pl.when(sid < NS - 1)
    def _():
        pltpu.sync_copy(h_hbm.at[pl.ds(sid * _HSTRIPE, _HSTRIPE)],
                        tab_sh.at[pl.ds(sid * _HSTRIPE, _HSTRIPE)])

    @pl.when(sid == NS - 1)
    def _():
        pltpu.sync_copy(h_hbm.at[pl.ds((NS - 1) * _HSTRIPE, N - (NS - 1) * _HSTRIPE)],
                        tab_sh.at[pl.ds((NS - 1) * _HSTRIPE, N - (NS - 1) * _HSTRIPE)])

    pltpu.sync_copy(idx_hbm.at[pl.ds(wid * kpw, kpw)], idx_v)
    plsc.subcore_barrier()

    def fg(c, buf, sem):
        pltpu.async_copy(tab_sh.at[idx_v.at[c]], buf, sem)

    def wg(buf, sem):
        pltpu.make_async_copy(tab_sh.at[pl.ds(0, CH)], buf, sem).wait()

    def fw(c, buf, sem):
        pltpu.async_copy(buf, out_hbm.at[pl.ds((wid * kpw + c) * CH, CH)], sem)

    def ww(buf, sem):
        pltpu.make_async_copy(buf, out_hbm.at[pl.ds(0, CH)], sem).wait()

    ng = kpw // 2
    fg(0, b0, sg0)

    def body(k, _):
        c = 2 * k
        wg(b0, sg0)

        @pl.when(k > 0)
        def _():
            ww(b1, sw1)

        fg(c + 1, b1, sg1)
        fw(c, b0, sw0)
        wg(b1, sg1)
        ww(b0, sw0)

        @pl.when(k < ng - 1)
        def _():
            fg(c + 2, b0, sg0)

        fw(c + 1, b1, sw1)
        return 0

    lax.fori_loop(0, ng, body, 0)
    ww(b1, sw1)


def _sc_gather(h, idx2, kpw):
    return pl.kernel(
        functools.partial(_gather_body, kpw=kpw),
        out_type=jax.ShapeDtypeStruct((NW * kpw * CH, D), jnp.float32),
        mesh=_sc_mesh(),
        scratch_types=[
            pltpu.VMEM((kpw, CH), jnp.int32),
            pltpu.VMEM((CH, D), jnp.float32),
            pltpu.VMEM((CH, D), jnp.float32),
            pltpu.VMEM_SHARED((N, D), jnp.float32),
            pltpu.SemaphoreType.DMA,
            pltpu.SemaphoreType.DMA,
            pltpu.SemaphoreType.DMA,
            pltpu.SemaphoreType.DMA,
        ],
    )(h, idx2)


def _scatter_body(m_hbm, idx_hbm, out_hbm, idx_v, m_b0, m_b1,
                  z_v, acc_sh, sr0, sr1, ss0, ss1, *, kpw):
    cid = lax.axis_index("c")
    sid = lax.axis_index("s")
    wid = sid * NC + cid
    pltpu.sync_copy(idx_hbm.at[pl.ds(wid * kpw, kpw)], idx_v)
    zrow = jnp.zeros((16,), jnp.float32)
    for r in range(8):
        for c in range(8):
            z_v[r, pl.ds(c * 16, 16)] = zrow

    def zbody(k, _):
        pltpu.sync_copy(z_v, acc_sh.at[pl.ds(sid * RPT + k * 8, 8)])
        return 0

    lax.fori_loop(0, RPT // 8, zbody, 0)
    plsc.subcore_barrier()

    def fr(c, buf, sem):
        pltpu.async_copy(m_hbm.at[pl.ds((wid * kpw + c) * CH, CH)], buf, sem)

    def wr(buf, sem):
        pltpu.make_async_copy(m_hbm.at[pl.ds(0, CH)], buf, sem).wait()

    def fs(c, buf, sem):
        pltpu.async_copy(buf, acc_sh.at[idx_v.at[c]], sem, add=True)

    def ws(buf, sem):
        pltpu.make_async_copy(buf, acc_sh.at[pl.ds(0, CH)], sem).wait()

    ng = kpw // 2
    fr(0, m_b0, sr0)

    def sbody(k, _):
        c = 2 * k
        wr(m_b0, sr0)

        @pl.when(k > 0)
        def _():
            ws(m_b1, ss1)

        fr(c + 1, m_b1, sr1)
        fs(c, m_b0, ss0)
        wr(m_b1, sr1)
        ws(m_b0, ss0)

        @pl.when(k < ng - 1)
        def _():
            fr(c + 2, m_b0, sr0)

        fs(c + 1, m_b1, ss1)
        return 0

    lax.fori_loop(0, ng, sbody, 0)
    ws(m_b1, ss1)
    plsc.subcore_barrier()
    pltpu.sync_copy(acc_sh.at[pl.ds(sid * RPT, RPT)],
                    out_hbm.at[cid, pl.ds(sid * RPT, RPT)])


def _sc_scatter(m, idx2, kpw):
    return pl.kernel(
        functools.partial(_scatter_body, kpw=kpw),
        out_type=jax.ShapeDtypeStruct((NC, N_PAD, D), jnp.float32),
        mesh=_sc_mesh(),
        scratch_types=[
            pltpu.VMEM((kpw, CH), jnp.int32),
            pltpu.VMEM((CH, D), jnp.float32),
            pltpu.VMEM((CH, D), jnp.float32),
            pltpu.VMEM((8, D), jnp.float32),
            pltpu.VMEM_SHARED((N_PAD, D), jnp.float32),
            pltpu.SemaphoreType.DMA,
            pltpu.SemaphoreType.DMA,
            pltpu.SemaphoreType.DMA,
            pltpu.SemaphoreType.DMA,
        ],
    )(m, idx2)


def kernel(x, edge_index, edge_attr, batch, emb_W, emb_b, emb_g, emb_beta,
           msg_W1, msg_b1, msg_g1, msg_be1, msg_W2, msg_b2,
           upd_W, upd_b, upd_g, upd_be, comb_W, comb_b, comb_g, comb_be):
    row = edge_index[0].astype(jnp.int32)
    col = edge_index[1].astype(jnp.int32)
    npad = E_PAD - E
    row_pad = jnp.concatenate([row, jnp.zeros((npad,), jnp.int32)])
    dummy = N + (lax.iota(jnp.int32, npad) % (N_PAD - N))
    col_pad = jnp.concatenate([col, dummy])
    ea_pad = jnp.concatenate([edge_attr, jnp.zeros((npad, ED), jnp.float32)],
                             axis=0).astype(jnp.bfloat16)
    row2s = [row_pad[s0 * E_S:(s0 + 1) * E_S].reshape(NW * KPW_S, CH)
             for s0 in range(S)]
    col2s = [col_pad[s0 * E_S:(s0 + 1) * E_S].reshape(NW * KPW_S, CH)
             for s0 in range(S)]
    eas = [ea_pad[s0 * E_S:(s0 + 1) * E_S] for s0 in range(S)]
    batch3 = batch.astype(jnp.int32).reshape(NBN, 1, BN)

    h, counts = _embed_call(x, emb_W, emb_b.reshape(1, D), emb_g.reshape(1, D),
                            emb_beta.reshape(1, D), batch3)
    reps = []
    for l in range(L):
        hgs = [_sc_gather(h, row2s[s0], KPW_S) for s0 in range(S)]
        ms = [_msg_call(hgs[s0], eas[s0], msg_W1[l, :D].astype(jnp.bfloat16),
                        msg_W1[l, D:].astype(jnp.bfloat16),
                        msg_b1[l].reshape(1, 2 * D), msg_g1[l].reshape(1, 2 * D),
                        msg_be1[l].reshape(1, 2 * D), msg_W2[l].astype(jnp.bfloat16),
                        msg_b2[l].reshape(1, D))
              for s0 in range(S)]
        pss = [_sc_scatter(ms[s0], col2s[s0], KPW_S) for s0 in range(S)]
        parts = jnp.concatenate(pss, axis=0)
        parts = lax.slice(parts, (0, 0, 0), (NC * S, N, D))
        h, ps = _upd_call(h, parts, upd_W[l, :D], upd_W[l, D:],
                          upd_b[l].reshape(1, D), upd_g[l].reshape(1, D),
                          upd_be[l].reshape(1, D), batch3)
        reps.append(ps)

    g = _comb_call(jnp.stack(reps), counts, comb_W.reshape(L, D, D),
                   comb_b.reshape(1, D), comb_g.reshape(1, D),
                   comb_be.reshape(1, D))
    return (g, h)


# speedup vs baseline: 1.1108x; 1.0010x over previous
import functools

import jax
import jax.numpy as jnp
from jax import lax
from jax.experimental import pallas as pl
from jax.experimental.pallas import tpu as pltpu
from jax.experimental.pallas import tpu_sc as plsc

N = 10000
E = 320000
D = 128
ED = 16
L = 4
G = 64

NC = 2
NS = 16
NW = NC * NS

CH = 128
KPW = 80
EPW = KPW * CH
E_PAD = NW * EPW
N_PAD = 10240
RPT = N_PAD // NS
ZR = 64

S = 2
KPW_S = KPW // S
E_S = E_PAD // S
BN = 2000
BE = 4096
NBN = N // BN


def _gelu(h):
    return 0.5 * h * (1.0 + lax.erf(h * 0.7071067811865476))


def _ln(h, g, b, eps=1e-5):
    mu = jnp.mean(h, axis=-1, keepdims=True)
    c = h - mu
    var = jnp.mean(c * c, axis=-1, keepdims=True)
    return c * jax.lax.rsqrt(var + eps) * g + b


def _onehot_t(batch_row, nrows):
    ids = lax.broadcasted_iota(jnp.int32, (G, nrows), 0)
    return (ids == jnp.broadcast_to(batch_row[None, :], (G, nrows))).astype(jnp.float32)


def _embed_body(x_ref, w_ref, b_ref, g_ref, be_ref, batch_ref, h_ref, cnt_ref):
    i = pl.program_id(0)
    h = jnp.dot(x_ref[...], w_ref[...], preferred_element_type=jnp.float32)
    h = _gelu(_ln(h + b_ref[...], g_ref[...], be_ref[...]))
    h_ref[...] = h
    oht = _onehot_t(batch_ref[0, 0, :], BN)
    cnt = jnp.sum(oht, axis=1, keepdims=True)

    @pl.when(i == 0)
    def _():
        cnt_ref[...] = jnp.zeros_like(cnt_ref)

    cnt_ref[...] += jnp.broadcast_to(cnt, (G, D))


def _embed_call(x, w, b, g, be, batch3):
    return pl.pallas_call(
        _embed_body,
        grid=(NBN,),
        in_specs=[
            pl.BlockSpec((BN, D), lambda i: (i, 0)),
            pl.BlockSpec((D, D), lambda i: (0, 0)),
            pl.BlockSpec((1, D), lambda i: (0, 0)),
            pl.BlockSpec((1, D), lambda i: (0, 0)),
            pl.BlockSpec((1, D), lambda i: (0, 0)),
            pl.BlockSpec((1, 1, BN), lambda i: (i, 0, 0)),
        ],
        out_specs=[
            pl.BlockSpec((BN, D), lambda i: (i, 0)),
            pl.BlockSpec((G, D), lambda i: (0, 0)),
        ],
        out_shape=[
            jax.ShapeDtypeStruct((N, D), jnp.float32),
            jax.ShapeDtypeStruct((G, D), jnp.float32),
        ],
    )(x, w, b, g, be, batch3)


def _msg_body(hg_ref, ea_ref, w1h_ref, w1e_ref, b1_ref, g1_ref, be1_ref,
              w2_ref, b2_ref, m_ref):
    m1 = jnp.dot(hg_ref[...], w1h_ref[...], preferred_element_type=jnp.float32)
    m1 = m1 + jnp.dot(ea_ref[...], w1e_ref[...], preferred_element_type=jnp.float32)
    m1 = _gelu(_ln(m1 + b1_ref[...], g1_ref[...], be1_ref[...]))
    m_ref[...] = jnp.dot(m1, w2_ref[...], preferred_element_type=jnp.float32) + b2_ref[...]


def _msg_call(hg, ea, w1h, w1e, b1, g1, be1, w2, b2):
    return pl.pallas_call(
        _msg_body,
        grid=(hg.shape[0] // BE,),
        in_specs=[
            pl.BlockSpec((BE, D), lambda i: (i, 0)),
            pl.BlockSpec((BE, ED), lambda i: (i, 0)),
            pl.BlockSpec((D, 2 * D), lambda i: (0, 0)),
            pl.BlockSpec((ED, 2 * D), lambda i: (0, 0)),
            pl.BlockSpec((1, 2 * D), lambda i: (0, 0)),
            pl.BlockSpec((1, 2 * D), lambda i: (0, 0)),
            pl.BlockSpec((1, 2 * D), lambda i: (0, 0)),
            pl.BlockSpec((2 * D, D), lambda i: (0, 0)),
            pl.BlockSpec((1, D), lambda i: (0, 0)),
        ],
        out_specs=pl.BlockSpec((BE, D), lambda i: (i, 0)),
        out_shape=jax.ShapeDtypeStruct((hg.shape[0], D), jnp.float32),
    )(hg, ea, w1h, w1e, b1, g1, be1, w2, b2)


def _upd_body(h_ref, p_ref, wh_ref, wa_ref, b_ref, g_ref, be_ref, batch_ref,
              hn_ref, ps_ref):
    i = pl.program_id(0)
    h = h_ref[...]
    agg = p_ref[0] + p_ref[1] + p_ref[2] + p_ref[3]
    u = jnp.dot(h, wh_ref[...], preferred_element_type=jnp.float32)
    u = u + jnp.dot(agg, wa_ref[...], preferred_element_type=jnp.float32)
    u = _gelu(_ln(u + b_ref[...], g_ref[...], be_ref[...]))
    hn = u + h
    hn_ref[...] = hn
    oht = _onehot_t(batch_ref[0, 0, :], BN)

    @pl.when(i == 0)
    def _():
        ps_ref[...] = jnp.zeros_like(ps_ref)

    ps_ref[...] += jnp.dot(oht, hn, preferred_element_type=jnp.float32)


def _upd_call(h, parts, wh, wa, b, g, be, batch3):
    return pl.pallas_call(
        _upd_body,
        grid=(NBN,),
        in_specs=[
            pl.BlockSpec((BN, D), lambda i: (i, 0)),
            pl.BlockSpec((2 * S, BN, D), lambda i: (0, i, 0)),
            pl.BlockSpec((D, D), lambda i: (0, 0)),
            pl.BlockSpec((D, D), lambda i: (0, 0)),
            pl.BlockSpec((1, D), lambda i: (0, 0)),
            pl.BlockSpec((1, D), lambda i: (0, 0)),
            pl.BlockSpec((1, D), lambda i: (0, 0)),
            pl.BlockSpec((1, 1, BN), lambda i: (i, 0, 0)),
        ],
        out_specs=[
            pl.BlockSpec((BN, D), lambda i: (i, 0)),
            pl.BlockSpec((G, D), lambda i: (0, 0)),
        ],
        out_shape=[
            jax.ShapeDtypeStruct((N, D), jnp.float32),
            jax.ShapeDtypeStruct((G, D), jnp.float32),
        ],
    )(h, parts, wh, wa, b, g, be, batch3)


def _comb_body(reps_ref, cnt_ref, w_ref, b_ref, g_ref, be_ref, out_ref):
    cnt = jnp.maximum(cnt_ref[...], 1.0)
    acc = jnp.zeros((G, D), jnp.float32)
    for l in range(L):
        acc = acc + jnp.dot(reps_ref[l] / cnt, w_ref[l],
                            preferred_element_type=jnp.float32)
    out_ref[...] = _gelu(_ln(acc + b_ref[...], g_ref[...], be_ref[...]))


def _comb_call(reps, cnt, w, b, g, be):
    return pl.pallas_call(
        _comb_body,
        out_shape=jax.ShapeDtypeStruct((G, D), jnp.float32),
    )(reps, cnt, w, b, g, be)


@functools.cache
def _sc_mesh():
    return plsc.VectorSubcoreMesh(core_axis_name="c", subcore_axis_name="s",
                                  num_cores=NC, num_subcores=NS)


_HSTRIPE = 632


def _gather_body(h_hbm, idx_hbm, out_hbm, idx_v, b0, b1, tab_sh,
                 sg0, sg1, sw0, sw1, *, kpw):
    cid = lax.axis_index("c")
    sid = lax.axis_index("s")
    wid = sid * NC + cid

    @pl.when(sid < NS - 1)
    def _():
        pltpu.sync_copy(h_hbm.at[pl.ds(sid * _HSTRIPE, _HSTRIPE)],
                        tab_sh.at[pl.ds(sid * _HSTRIPE, _HSTRIPE)])

    @pl.when(sid == NS - 1)
    def _():
        pltpu.sync_copy(h_hbm.at[pl.ds((NS - 1) * _HSTRIPE, N - (NS - 1) * _HSTRIPE)],
                        tab_sh.at[pl.ds((NS - 1) * _HSTRIPE, N - (NS - 1) * _HSTRIPE)])

    pltpu.sync_copy(idx_hbm.at[pl.ds(wid * kpw, kpw)], idx_v)
    plsc.subcore_barrier()

    def fg(c, buf, sem):
        pltpu.async_copy(tab_sh.at[idx_v.at[c]], buf, sem)

    def wg(buf, sem):
        pltpu.make_async_copy(tab_sh.at[pl.ds(0, CH)], buf, sem).wait()

    def fw(c, buf, sem):
        pltpu.async_copy(buf, out_hbm.at[pl.ds((wid * kpw + c) * CH, CH)], sem)

    def ww(buf, sem):
        pltpu.make_async_copy(buf, out_hbm.at[pl.ds(0, CH)], sem).wait()

    ng = kpw // 2
    fg(0, b0, sg0)

    def body(k, _):
        c = 2 * k
        wg(b0, sg0)

        @pl.when(k > 0)
        def _():
            ww(b1, sw1)

        fg(c + 1, b1, sg1)
        fw(c, b0, sw0)
        wg(b1, sg1)
        ww(b0, sw0)

        @pl.when(k < ng - 1)
        def _():
            fg(c + 2, b0, sg0)

        fw(c + 1, b1, sw1)
        return 0

    lax.fori_loop(0, ng, body, 0)
    ww(b1, sw1)


def _sc_gather(h, idx2, kpw):
    return pl.kernel(
        functools.partial(_gather_body, kpw=kpw),
        out_type=jax.ShapeDtypeStruct((NW * kpw * CH, D), jnp.float32),
        mesh=_sc_mesh(),
        scratch_types=[
            pltpu.VMEM((kpw, CH), jnp.int32),
            pltpu.VMEM((CH, D), jnp.float32),
            pltpu.VMEM((CH, D), jnp.float32),
            pltpu.VMEM_SHARED((N, D), jnp.float32),
            pltpu.SemaphoreType.DMA,
            pltpu.SemaphoreType.DMA,
            pltpu.SemaphoreType.DMA,
            pltpu.SemaphoreType.DMA,
        ],
    )(h, idx2)


def _scatter_body(m_hbm, idx_hbm, out_hbm, idx_v, m_b0, m_b1,
                  z_v, acc_sh, sr0, sr1, ss0, ss1, *, kpw):
    cid = lax.axis_index("c")
    sid = lax.axis_index("s")
    wid = sid * NC + cid
    pltpu.sync_copy(idx_hbm.at[pl.ds(wid * kpw, kpw)], idx_v)
    zrow = jnp.zeros((16,), jnp.float32)
    for r in range(ZR):
        for c in range(8):
            z_v[r, pl.ds(c * 16, 16)] = zrow

    def zfire(k, _):
        pltpu.async_copy(z_v, acc_sh.at[pl.ds(sid * RPT + k * ZR, ZR)], sr0)
        return 0

    def zdrain(k, _):
        pltpu.make_async_copy(z_v, acc_sh.at[pl.ds(sid * RPT, ZR)], sr0).wait()
        return 0

    lax.fori_loop(0, RPT // ZR, zfire, 0)
    lax.fori_loop(0, RPT // ZR, zdrain, 0)
    plsc.subcore_barrier()

    def fr(c, buf, sem):
        pltpu.async_copy(m_hbm.at[pl.ds((wid * kpw + c) * CH, CH)], buf, sem)

    def wr(buf, sem):
        pltpu.make_async_copy(m_hbm.at[pl.ds(0, CH)], buf, sem).wait()

    def fs(c, buf, sem):
        pltpu.async_copy(buf, acc_sh.at[idx_v.at[c]], sem, add=True)

    def ws(buf, sem):
        pltpu.make_async_copy(buf, acc_sh.at[pl.ds(0, CH)], sem).wait()

    ng = kpw // 2
    fr(0, m_b0, sr0)

    def sbody(k, _):
        c = 2 * k
        wr(m_b0, sr0)

        @pl.when(k > 0)
        def _():
            ws(m_b1, ss1)

        fr(c + 1, m_b1, sr1)
        fs(c, m_b0, ss0)
        wr(m_b1, sr1)
        ws(m_b0, ss0)

        @pl.when(k < ng - 1)
        def _():
            fr(c + 2, m_b0, sr0)

        fs(c + 1, m_b1, ss1)
        return 0

    lax.fori_loop(0, ng, sbody, 0)
    ws(m_b1, ss1)
    plsc.subcore_barrier()
    pltpu.sync_copy(acc_sh.at[pl.ds(sid * RPT, RPT)],
                    out_hbm.at[cid, pl.ds(sid * RPT, RPT)])


def _sc_scatter(m, idx2, kpw):
    return pl.kernel(
        functools.partial(_scatter_body, kpw=kpw),
        out_type=jax.ShapeDtypeStruct((NC, N_PAD, D), jnp.float32),
        mesh=_sc_mesh(),
        scratch_types=[
            pltpu.VMEM((kpw, CH), jnp.int32),
            pltpu.VMEM((CH, D), jnp.float32),
            pltpu.VMEM((CH, D), jnp.float32),
            pltpu.VMEM((ZR, D), jnp.float32),
            pltpu.VMEM_SHARED((N_PAD, D), jnp.float32),
            pltpu.SemaphoreType.DMA,
            pltpu.SemaphoreType.DMA,
            pltpu.SemaphoreType.DMA,
            pltpu.SemaphoreType.DMA,
        ],
    )(m, idx2)


def kernel(x, edge_index, edge_attr, batch, emb_W, emb_b, emb_g, emb_beta,
           msg_W1, msg_b1, msg_g1, msg_be1, msg_W2, msg_b2,
           upd_W, upd_b, upd_g, upd_be, comb_W, comb_b, comb_g, comb_be):
    row = edge_index[0].astype(jnp.int32)
    col = edge_index[1].astype(jnp.int32)
    npad = E_PAD - E
    row_pad = jnp.concatenate([row, jnp.zeros((npad,), jnp.int32)])
    dummy = N + (lax.iota(jnp.int32, npad) % (N_PAD - N))
    col_pad = jnp.concatenate([col, dummy])
    ea_pad = jnp.concatenate([edge_attr, jnp.zeros((npad, ED), jnp.float32)],
                             axis=0)
    row2s = [row_pad[s0 * E_S:(s0 + 1) * E_S].reshape(NW * KPW_S, CH)
             for s0 in range(S)]
    col2s = [col_pad[s0 * E_S:(s0 + 1) * E_S].reshape(NW * KPW_S, CH)
             for s0 in range(S)]
    eas = [ea_pad[s0 * E_S:(s0 + 1) * E_S] for s0 in range(S)]
    batch3 = batch.astype(jnp.int32).reshape(NBN, 1, BN)

    h, counts = _embed_call(x, emb_W, emb_b.reshape(1, D), emb_g.reshape(1, D),
                            emb_beta.reshape(1, D), batch3)
    reps = []
    for l in range(L):
        hgs = [_sc_gather(h, row2s[s0], KPW_S) for s0 in range(S)]
        ms = [_msg_call(hgs[s0], eas[s0], msg_W1[l, :D], msg_W1[l, D:],
                        msg_b1[l].reshape(1, 2 * D), msg_g1[l].reshape(1, 2 * D),
                        msg_be1[l].reshape(1, 2 * D), msg_W2[l],
                        msg_b2[l].reshape(1, D))
              for s0 in range(S)]
        pss = [_sc_scatter(ms[s0], col2s[s0], KPW_S) for s0 in range(S)]
        parts = jnp.concatenate(pss, axis=0)
        parts = lax.slice(parts, (0, 0, 0), (NC * S, N, D))
        h, ps = _upd_call(h, parts, upd_W[l, :D], upd_W[l, D:],
                          upd_b[l].reshape(1, D), upd_g[l].reshape(1, D),
                          upd_be[l].reshape(1, D), batch3)
        reps.append(ps)

    g = _comb_call(jnp.stack(reps), counts, comb_W.reshape(L, D, D),
                   comb_b.reshape(1, D), comb_g.reshape(1, D),
                   comb_be.reshape(1, D))
    return (g, h)


# speedup vs baseline: 1.1125x; 1.0015x over previous
import functools

import jax
import jax.numpy as jnp
from jax import lax
from jax.experimental import pallas as pl
from jax.experimental.pallas import tpu as pltpu
from jax.experimental.pallas import tpu_sc as plsc

N = 10000
E = 320000
D = 128
ED = 16
L = 4
G = 64

NC = 2
NS = 16
NW = NC * NS

CH = 128
KPW = 80
EPW = KPW * CH
E_PAD = NW * EPW
N_PAD = 10240
RPT = N_PAD // NS
ZR = 64

S = 4
KPW_S = KPW // S
E_S = E_PAD // S
BN = 2000
BE = 4096
NBN = N // BN


def _gelu(h):
    return 0.5 * h * (1.0 + lax.erf(h * 0.7071067811865476))


def _ln(h, g, b, eps=1e-5):
    mu = jnp.mean(h, axis=-1, keepdims=True)
    c = h - mu
    var = jnp.mean(c * c, axis=-1, keepdims=True)
    return c * jax.lax.rsqrt(var + eps) * g + b


def _onehot_t(batch_row, nrows):
    ids = lax.broadcasted_iota(jnp.int32, (G, nrows), 0)
    return (ids == jnp.broadcast_to(batch_row[None, :], (G, nrows))).astype(jnp.float32)


def _embed_body(x_ref, w_ref, b_ref, g_ref, be_ref, batch_ref, h_ref, cnt_ref):
    i = pl.program_id(0)
    h = jnp.dot(x_ref[...], w_ref[...], preferred_element_type=jnp.float32)
    h = _gelu(_ln(h + b_ref[...], g_ref[...], be_ref[...]))
    h_ref[...] = h
    oht = _onehot_t(batch_ref[0, 0, :], BN)
    cnt = jnp.sum(oht, axis=1, keepdims=True)

    @pl.when(i == 0)
    def _():
        cnt_ref[...] = jnp.zeros_like(cnt_ref)

    cnt_ref[...] += jnp.broadcast_to(cnt, (G, D))


def _embed_call(x, w, b, g, be, batch3):
    return pl.pallas_call(
        _embed_body,
        grid=(NBN,),
        in_specs=[
            pl.BlockSpec((BN, D), lambda i: (i, 0)),
            pl.BlockSpec((D, D), lambda i: (0, 0)),
            pl.BlockSpec((1, D), lambda i: (0, 0)),
            pl.BlockSpec((1, D), lambda i: (0, 0)),
            pl.BlockSpec((1, D), lambda i: (0, 0)),
            pl.BlockSpec((1, 1, BN), lambda i: (i, 0, 0)),
        ],
        out_specs=[
            pl.BlockSpec((BN, D), lambda i: (i, 0)),
            pl.BlockSpec((G, D), lambda i: (0, 0)),
        ],
        out_shape=[
            jax.ShapeDtypeStruct((N, D), jnp.float32),
            jax.ShapeDtypeStruct((G, D), jnp.float32),
        ],
    )(x, w, b, g, be, batch3)


def _msg_body(hg_ref, ea_ref, w1h_ref, w1e_ref, b1_ref, g1_ref, be1_ref,
              w2_ref, b2_ref, m_ref):
    m1 = jnp.dot(hg_ref[...], w1h_ref[...], preferred_element_type=jnp.float32)
    m1 = m1 + jnp.dot(ea_ref[...], w1e_ref[...], preferred_element_type=jnp.float32)
    m1 = _gelu(_ln(m1 + b1_ref[...], g1_ref[...], be1_ref[...]))
    m_ref[...] = jnp.dot(m1, w2_ref[...], preferred_element_type=jnp.float32) + b2_ref[...]


def _msg_call(hg, ea, w1h, w1e, b1, g1, be1, w2, b2):
    return pl.pallas_call(
        _msg_body,
        grid=(hg.shape[0] // BE,),
        in_specs=[
            pl.BlockSpec((BE, D), lambda i: (i, 0)),
            pl.BlockSpec((BE, ED), lambda i: (i, 0)),
            pl.BlockSpec((D, 2 * D), lambda i: (0, 0)),
            pl.BlockSpec((ED, 2 * D), lambda i: (0, 0)),
            pl.BlockSpec((1, 2 * D), lambda i: (0, 0)),
            pl.BlockSpec((1, 2 * D), lambda i: (0, 0)),
            pl.BlockSpec((1, 2 * D), lambda i: (0, 0)),
            pl.BlockSpec((2 * D, D), lambda i: (0, 0)),
            pl.BlockSpec((1, D), lambda i: (0, 0)),
        ],
        out_specs=pl.BlockSpec((BE, D), lambda i: (i, 0)),
        out_shape=jax.ShapeDtypeStruct((hg.shape[0], D), jnp.float32),
    )(hg, ea, w1h, w1e, b1, g1, be1, w2, b2)


def _upd_body(h_ref, p_ref, wh_ref, wa_ref, b_ref, g_ref, be_ref, batch_ref,
              hn_ref, ps_ref):
    i = pl.program_id(0)
    h = h_ref[...]
    agg = p_ref[0]
    for q in range(1, 2 * S):
        agg = agg + p_ref[q]
    u = jnp.dot(h, wh_ref[...], preferred_element_type=jnp.float32)
    u = u + jnp.dot(agg, wa_ref[...], preferred_element_type=jnp.float32)
    u = _gelu(_ln(u + b_ref[...], g_ref[...], be_ref[...]))
    hn = u + h
    hn_ref[...] = hn
    oht = _onehot_t(batch_ref[0, 0, :], BN)

    @pl.when(i == 0)
    def _():
        ps_ref[...] = jnp.zeros_like(ps_ref)

    ps_ref[...] += jnp.dot(oht, hn, preferred_element_type=jnp.float32)


def _upd_call(h, parts, wh, wa, b, g, be, batch3):
    return pl.pallas_call(
        _upd_body,
        grid=(NBN,),
        in_specs=[
            pl.BlockSpec((BN, D), lambda i: (i, 0)),
            pl.BlockSpec((2 * S, BN, D), lambda i: (0, i, 0)),
            pl.BlockSpec((D, D), lambda i: (0, 0)),
            pl.BlockSpec((D, D), lambda i: (0, 0)),
            pl.BlockSpec((1, D), lambda i: (0, 0)),
            pl.BlockSpec((1, D), lambda i: (0, 0)),
            pl.BlockSpec((1, D), lambda i: (0, 0)),
            pl.BlockSpec((1, 1, BN), lambda i: (i, 0, 0)),
        ],
        out_specs=[
            pl.BlockSpec((BN, D), lambda i: (i, 0)),
            pl.BlockSpec((G, D), lambda i: (0, 0)),
        ],
        out_shape=[
            jax.ShapeDtypeStruct((N, D), jnp.float32),
            jax.ShapeDtypeStruct((G, D), jnp.float32),
        ],
    )(h, parts, wh, wa, b, g, be, batch3)


def _comb_body(reps_ref, cnt_ref, w_ref, b_ref, g_ref, be_ref, out_ref):
    cnt = jnp.maximum(cnt_ref[...], 1.0)
    acc = jnp.zeros((G, D), jnp.float32)
    for l in range(L):
        acc = acc + jnp.dot(reps_ref[l] / cnt, w_ref[l],
                            preferred_element_type=jnp.float32)
    out_ref[...] = _gelu(_ln(acc + b_ref[...], g_ref[...], be_ref[...]))


def _comb_call(reps, cnt, w, b, g, be):
    return pl.pallas_call(
        _comb_body,
        out_shape=jax.ShapeDtypeStruct((G, D), jnp.float32),
    )(reps, cnt, w, b, g, be)


@functools.cache
def _sc_mesh():
    return plsc.VectorSubcoreMesh(core_axis_name="c", subcore_axis_name="s",
                                  num_cores=NC, num_subcores=NS)


_HSTRIPE = 632


def _gather_body(h_hbm, idx_hbm, out_hbm, idx_v, b0, b1, tab_sh,
                 sg0, sg1, sw0, sw1, *, kpw):
    cid = lax.axis_index("c")
    sid = lax.axis_index("s")
    wid = sid * NC + cid

    @pl.when(sid < NS - 1)
    def _():
        pltpu.sync_copy(h_hbm.at[pl.ds(sid * _HSTRIPE, _HSTRIPE)],
                        tab_sh.at[pl.ds(sid * _HSTRIPE, _HSTRIPE)])

    @pl.when(sid == NS - 1)
    def _():
        pltpu.sync_copy(h_hbm.at[pl.ds((NS - 1) * _HSTRIPE, N - (NS - 1) * _HSTRIPE)],
                        tab_sh.at[pl.ds((NS - 1) * _HSTRIPE, N - (NS - 1) * _HSTRIPE)])

    pltpu.sync_copy(idx_hbm.at[wid], idx_v)
    plsc.subcore_barrier()

    def fg(c, buf, sem):
        pltpu.async_copy(tab_sh.at[idx_v.at[c]], buf, sem)

    def wg(buf, sem):
        pltpu.make_async_copy(tab_sh.at[pl.ds(0, CH)], buf, sem).wait()

    def fw(c, buf, sem):
        pltpu.async_copy(buf, out_hbm.at[pl.ds((wid * kpw + c) * CH, CH)], sem)

    def ww(buf, sem):
        pltpu.make_async_copy(buf, out_hbm.at[pl.ds(0, CH)], sem).wait()

    ng = kpw // 2
    fg(0, b0, sg0)

    def body(k, _):
        c = 2 * k
        wg(b0, sg0)

        @pl.when(k > 0)
        def _():
            ww(b1, sw1)

        fg(c + 1, b1, sg1)
        fw(c, b0, sw0)
        wg(b1, sg1)
        ww(b0, sw0)

        @pl.when(k < ng - 1)
        def _():
            fg(c + 2, b0, sg0)

        fw(c + 1, b1, sw1)
        return 0

    lax.fori_loop(0, ng, body, 0)
    ww(b1, sw1)


def _sc_gather(h, idx2, kpw):
    return pl.kernel(
        functools.partial(_gather_body, kpw=kpw),
        out_type=jax.ShapeDtypeStruct((NW * kpw * CH, D), jnp.float32),
        mesh=_sc_mesh(),
        scratch_types=[
            pltpu.VMEM((kpw, CH), jnp.int32),
            pltpu.VMEM((CH, D), jnp.float32),
            pltpu.VMEM((CH, D), jnp.float32),
            pltpu.VMEM_SHARED((N, D), jnp.float32),
            pltpu.SemaphoreType.DMA,
            pltpu.SemaphoreType.DMA,
            pltpu.SemaphoreType.DMA,
            pltpu.SemaphoreType.DMA,
        ],
    )(h, idx2)


def _scatter_body(m_hbm, idx_hbm, out_hbm, idx_v, m_b0, m_b1,
                  z_v, acc_sh, sr0, sr1, ss0, ss1, *, kpw):
    cid = lax.axis_index("c")
    sid = lax.axis_index("s")
    wid = sid * NC + cid
    pltpu.sync_copy(idx_hbm.at[wid], idx_v)
    zrow = jnp.zeros((16,), jnp.float32)
    for r in range(ZR):
        for c in range(8):
            z_v[r, pl.ds(c * 16, 16)] = zrow

    def zfire(k, _):
        pltpu.async_copy(z_v, acc_sh.at[pl.ds(sid * RPT + k * ZR, ZR)], sr0)
        return 0

    def zdrain(k, _):
        pltpu.make_async_copy(z_v, acc_sh.at[pl.ds(sid * RPT, ZR)], sr0).wait()
        return 0

    lax.fori_loop(0, RPT // ZR, zfire, 0)
    lax.fori_loop(0, RPT // ZR, zdrain, 0)
    plsc.subcore_barrier()

    def fr(c, buf, sem):
        pltpu.async_copy(m_hbm.at[pl.ds((wid * kpw + c) * CH, CH)], buf, sem)

    def wr(buf, sem):
        pltpu.make_async_copy(m_hbm.at[pl.ds(0, CH)], buf, sem).wait()

    def fs(c, buf, sem):
        pltpu.async_copy(buf, acc_sh.at[idx_v.at[c]], sem, add=True)

    def ws(buf, sem):
        pltpu.make_async_copy(buf, acc_sh.at[pl.ds(0, CH)], sem).wait()

    ng = kpw // 2
    fr(0, m_b0, sr0)

    def sbody(k, _):
        c = 2 * k
        wr(m_b0, sr0)

        @pl.when(k > 0)
        def _():
            ws(m_b1, ss1)

        fr(c + 1, m_b1, sr1)
        fs(c, m_b0, ss0)
        wr(m_b1, sr1)
        ws(m_b0, ss0)

        @pl.when(k < ng - 1)
        def _():
            fr(c + 2, m_b0, sr0)

        fs(c + 1, m_b1, ss1)
        return 0

    lax.fori_loop(0, ng, sbody, 0)
    ws(m_b1, ss1)
    plsc.subcore_barrier()
    pltpu.sync_copy(acc_sh.at[pl.ds(sid * RPT, RPT)],
                    out_hbm.at[cid, pl.ds(sid * RPT, RPT)])


def _sc_scatter(m, idx2, kpw):
    return pl.kernel(
        functools.partial(_scatter_body, kpw=kpw),
        out_type=jax.ShapeDtypeStruct((NC, N_PAD, D), jnp.float32),
        mesh=_sc_mesh(),
        scratch_types=[
            pltpu.VMEM((kpw, CH), jnp.int32),
            pltpu.VMEM((CH, D), jnp.float32),
            pltpu.VMEM((CH, D), jnp.float32),
            pltpu.VMEM((ZR, D), jnp.float32),
            pltpu.VMEM_SHARED((N_PAD, D), jnp.float32),
            pltpu.SemaphoreType.DMA,
            pltpu.SemaphoreType.DMA,
            pltpu.SemaphoreType.DMA,
            pltpu.SemaphoreType.DMA,
        ],
    )(m, idx2)


def kernel(x, edge_index, edge_attr, batch, emb_W, emb_b, emb_g, emb_beta,
           msg_W1, msg_b1, msg_g1, msg_be1, msg_W2, msg_b2,
           upd_W, upd_b, upd_g, upd_be, comb_W, comb_b, comb_g, comb_be):
    row = edge_index[0].astype(jnp.int32)
    col = edge_index[1].astype(jnp.int32)
    npad = E_PAD - E
    row_pad = jnp.concatenate([row, jnp.zeros((npad,), jnp.int32)])
    dummy = N + (lax.iota(jnp.int32, npad) % (N_PAD - N))
    col_pad = jnp.concatenate([col, dummy])
    ea_pad = jnp.concatenate([edge_attr, jnp.zeros((npad, ED), jnp.float32)],
                             axis=0)
    row2s = [row_pad[s0 * E_S:(s0 + 1) * E_S].reshape(NW, KPW_S, CH)
             for s0 in range(S)]
    col2s = [col_pad[s0 * E_S:(s0 + 1) * E_S].reshape(NW, KPW_S, CH)
             for s0 in range(S)]
    eas = [ea_pad[s0 * E_S:(s0 + 1) * E_S] for s0 in range(S)]
    batch3 = batch.astype(jnp.int32).reshape(NBN, 1, BN)

    h, counts = _embed_call(x, emb_W, emb_b.reshape(1, D), emb_g.reshape(1, D),
                            emb_beta.reshape(1, D), batch3)
    reps = []
    for l in range(L):
        hgs = [_sc_gather(h, row2s[s0], KPW_S) for s0 in range(S)]
        ms = [_msg_call(hgs[s0], eas[s0], msg_W1[l, :D], msg_W1[l, D:],
                        msg_b1[l].reshape(1, 2 * D), msg_g1[l].reshape(1, 2 * D),
                        msg_be1[l].reshape(1, 2 * D), msg_W2[l],
                        msg_b2[l].reshape(1, D))
              for s0 in range(S)]
        pss = [_sc_scatter(ms[s0], col2s[s0], KPW_S) for s0 in range(S)]
        parts = jnp.concatenate(pss, axis=0)
        parts = lax.slice(parts, (0, 0, 0), (NC * S, N, D))
        h, ps = _upd_call(h, parts, upd_W[l, :D], upd_W[l, D:],
                          upd_b[l].reshape(1, D), upd_g[l].reshape(1, D),
                          upd_be[l].reshape(1, D), batch3)
        reps.append(ps)

    g = _comb_call(jnp.stack(reps), counts, comb_W.reshape(L, D, D),
                   comb_b.reshape(1, D), comb_g.reshape(1, D),
                   comb_be.reshape(1, D))
    return (g, h)
